# baseline (speedup 1.0000x reference)
import functools

import jax
import jax.numpy as jnp
from jax import lax
from jax.experimental import pallas as pl
from jax.experimental.pallas import tpu as pltpu
from jax.experimental.pallas import tpu_sc as plsc

B = 16384
L = 200
EMB = 32
HID = 128
OUT = 2

NC = 2
NS = 16
NW = NC * NS
SPW = B // NW
C = 8
NCHUNK = SPW // C
NPAIR = NCHUNK // 2
CI = C * L


def _pool_body(
    text_ref, table_ref, out_ref, idx_v, rows_v, pool_v, sem0, sem1, osem0, osem1
):
    cid = lax.axis_index("c")
    sid = lax.axis_index("s")
    wid = sid * NC + cid
    sample0 = wid * SPW
    sems = (sem0, sem1)
    osems = (osem0, osem1)

    def issue(ch, b):
        samp = sample0 + ch * C
        pltpu.sync_copy(text_ref.at[pl.ds(samp, C)], idx_v.at[b])
        for sloc in range(C):
            rb = sloc * L
            pltpu.async_copy(
                table_ref.at[idx_v.at[b, sloc, pl.ds(0, 128)]],
                rows_v.at[b, pl.ds(rb, 128)],
                sems[b],
            )
            pltpu.async_copy(
                table_ref.at[idx_v.at[b, sloc, pl.ds(128, 72)]],
                rows_v.at[b, pl.ds(rb + 128, 72)],
                sems[b],
            )

    def drain(b):
        pltpu.make_async_copy(
            table_ref.at[pl.ds(0, CI)], rows_v.at[b], sems[b]
        ).wait()

    def accumulate(ch, b):
        samp = sample0 + ch * C
        @pl.when(ch >= 2)
        def _():
            pltpu.make_async_copy(
                pool_v.at[b], out_ref.at[pl.ds(samp, C)], osems[b]
            ).wait()

        hi_mask = jnp.full((16,), -65536, jnp.int32)

        for sloc in range(C):
            rbase = sloc * L

            def rbody(j, acc, rbase=rbase, b=b):
                aee, aeo, aoe, aoo = acc
                r = rbase + j * 8
                for k in range(0, 8, 2):
                    x0 = plsc.bitcast(rows_v[b, r + k, pl.ds(0, 32)], jnp.int32)
                    x1 = plsc.bitcast(rows_v[b, r + k + 1, pl.ds(0, 32)], jnp.int32)
                    aee = aee + plsc.bitcast(lax.shift_left(x0, 16), jnp.float32)
                    aeo = aeo + plsc.bitcast(x0 & hi_mask, jnp.float32)
                    aoe = aoe + plsc.bitcast(lax.shift_left(x1, 16), jnp.float32)
                    aoo = aoo + plsc.bitcast(x1 & hi_mask, jnp.float32)
                return (aee, aeo, aoe, aoo)

            z = jnp.zeros((16,), jnp.float32)
            aee, aeo, aoe, aoo = lax.fori_loop(0, L // 8, rbody, (z, z, z, z))
            pool_v[b, sloc, pl.ds(0, 16)] = aee + aoe
            pool_v[b, sloc, pl.ds(16, 16)] = aeo + aoo
        pltpu.async_copy(pool_v.at[b], out_ref.at[pl.ds(samp, C)], osems[b])

    issue(0, 0)

    def pair_body(p, carry):
        ch0 = p * 2
        issue(ch0 + 1, 1)
        drain(0)
        accumulate(ch0, 0)

        @pl.when(p < NPAIR - 1)
        def _():
            issue(ch0 + 2, 0)

        drain(1)
        accumulate(ch0 + 1, 1)
        return carry

    lax.fori_loop(0, NPAIR, pair_body, 0)

    for b in (0, 1):
        pltpu.make_async_copy(
            pool_v.at[b], out_ref.at[pl.ds(sample0, C)], osems[b]
        ).wait()


_pool = functools.partial(
    pl.kernel,
    out_type=jax.ShapeDtypeStruct((B, EMB), jnp.float32),
    mesh=plsc.VectorSubcoreMesh(core_axis_name="c", subcore_axis_name="s"),
    scratch_types=[
        pltpu.VMEM((2, C, L), jnp.int32),
        pltpu.VMEM((2, CI, EMB), jnp.bfloat16),
        pltpu.VMEM((2, C, EMB), jnp.float32),
        pltpu.SemaphoreType.DMA,
        pltpu.SemaphoreType.DMA,
        pltpu.SemaphoreType.DMA,
        pltpu.SemaphoreType.DMA,
    ],
    compiler_params=pltpu.CompilerParams(
        use_tc_tiling_on_sc=False, needs_layout_passes=False
    ),
)(_pool_body)


BM = 2048


def _mlp_body(x_ref, w1_ref, b1_ref, w2_ref, b2_ref, w3_ref, b3_ref, o_ref):
    x = x_ref[...]
    h = jnp.dot(x, w1_ref[...], preferred_element_type=jnp.float32) + b1_ref[...]
    h = jnp.maximum(h, 0.0)
    h = jnp.dot(h, w2_ref[...], preferred_element_type=jnp.float32) + b2_ref[...]
    h = jnp.maximum(h, 0.0)
    o_ref[...] = (
        jnp.dot(h, w3_ref[...], preferred_element_type=jnp.float32) + b3_ref[...]
    )


def _mlp(pooled, W1s, b1, W2, b2, W3, b3):
    return pl.pallas_call(
        _mlp_body,
        grid=(B // BM,),
        in_specs=[
            pl.BlockSpec((BM, EMB), lambda i: (i, 0)),
            pl.BlockSpec((EMB, HID), lambda i: (0, 0)),
            pl.BlockSpec((1, HID), lambda i: (0, 0)),
            pl.BlockSpec((HID, HID), lambda i: (0, 0)),
            pl.BlockSpec((1, HID), lambda i: (0, 0)),
            pl.BlockSpec((HID, OUT), lambda i: (0, 0)),
            pl.BlockSpec((1, OUT), lambda i: (0, 0)),
        ],
        out_specs=pl.BlockSpec((BM, OUT), lambda i: (i, 0)),
        out_shape=jax.ShapeDtypeStruct((B, OUT), jnp.float32),
    )(pooled, W1s, b1, W2, b2, W3, b3)


_W1_PERM = [2 * i for i in range(16)] + [2 * i + 1 for i in range(16)]


def kernel(text, table, W1, b1, W2, b2, W3, b3):
    pooled_sum = _pool(text, table.astype(jnp.bfloat16))
    W1s = W1[jnp.array(_W1_PERM), :] * jnp.float32(1.0 / L)
    return _mlp(
        pooled_sum,
        W1s,
        b1.reshape(1, HID),
        W2,
        b2.reshape(1, HID),
        W3,
        b3.reshape(1, OUT),
    )


# speedup vs baseline: 1.0895x; 1.0895x over previous
import functools

import jax
import jax.numpy as jnp
from jax import lax
from jax.experimental import pallas as pl
from jax.experimental.pallas import tpu as pltpu
from jax.experimental.pallas import tpu_sc as plsc

B = 16384
L = 200
EMB = 32
HID = 128
OUT = 2

NC = 2
NS = 16
NW = NC * NS
SPW = B // NW
C = 8
NCHUNK = SPW // C
NPAIR = NCHUNK // 2
CI = C * L


def _pool_body(
    text_ref, table_ref, out_ref, idx_v, rows_v, pool_v, sem0, sem1, osem0, osem1
):
    cid = lax.axis_index("c")
    sid = lax.axis_index("s")
    wid = sid * NC + cid
    sample0 = wid * SPW
    sems = (sem0, sem1)
    osems = (osem0, osem1)

    def issue(ch, b):
        samp = sample0 + ch * C
        pltpu.sync_copy(text_ref.at[pl.ds(samp, C)], idx_v.at[b])
        for sloc in range(C):
            rb = sloc * L
            pltpu.async_copy(
                table_ref.at[idx_v.at[b, sloc]],
                rows_v.at[b, pl.ds(rb, L)],
                sems[b],
            )

    def drain(b):
        pltpu.make_async_copy(
            table_ref.at[pl.ds(0, CI)], rows_v.at[b], sems[b]
        ).wait()

    def accumulate(ch, b):
        samp = sample0 + ch * C
        @pl.when(ch >= 2)
        def _():
            pltpu.make_async_copy(
                pool_v.at[b], out_ref.at[pl.ds(samp, C)], osems[b]
            ).wait()

        for sloc in range(C):
            rbase = sloc * L

            def rbody(j, acc, rbase=rbase, b=b):
                e0, e1, o0, o1 = acc
                r = rbase + j * 8
                for k in range(0, 8, 2):
                    e0 = e0 + rows_v[b, r + k, pl.ds(0, 16)]
                    e1 = e1 + rows_v[b, r + k, pl.ds(16, 16)]
                    o0 = o0 + rows_v[b, r + k + 1, pl.ds(0, 16)]
                    o1 = o1 + rows_v[b, r + k + 1, pl.ds(16, 16)]
                return (e0, e1, o0, o1)

            z = jnp.zeros((16,), jnp.float32)
            e0, e1, o0, o1 = lax.fori_loop(0, L // 8, rbody, (z, z, z, z))
            pool_v[b, sloc, pl.ds(0, 16)] = e0 + o0
            pool_v[b, sloc, pl.ds(16, 16)] = e1 + o1
        pltpu.async_copy(pool_v.at[b], out_ref.at[pl.ds(samp, C)], osems[b])

    issue(0, 0)

    def pair_body(p, carry):
        ch0 = p * 2
        issue(ch0 + 1, 1)
        drain(0)
        accumulate(ch0, 0)

        @pl.when(p < NPAIR - 1)
        def _():
            issue(ch0 + 2, 0)

        drain(1)
        accumulate(ch0 + 1, 1)
        return carry

    lax.fori_loop(0, NPAIR, pair_body, 0)

    for b in (0, 1):
        pltpu.make_async_copy(
            pool_v.at[b], out_ref.at[pl.ds(sample0, C)], osems[b]
        ).wait()


_pool = functools.partial(
    pl.kernel,
    out_type=jax.ShapeDtypeStruct((B, EMB), jnp.float32),
    mesh=plsc.VectorSubcoreMesh(core_axis_name="c", subcore_axis_name="s"),
    scratch_types=[
        pltpu.VMEM((2, C, L), jnp.int32),
        pltpu.VMEM((2, CI, EMB), jnp.float32),
        pltpu.VMEM((2, C, EMB), jnp.float32),
        pltpu.SemaphoreType.DMA,
        pltpu.SemaphoreType.DMA,
        pltpu.SemaphoreType.DMA,
        pltpu.SemaphoreType.DMA,
    ],
    compiler_params=pltpu.CompilerParams(
        use_tc_tiling_on_sc=False, needs_layout_passes=False
    ),
)(_pool_body)


BM = 2048


def _mlp_body(x_ref, w1_ref, b1_ref, w2_ref, b2_ref, w3_ref, b3_ref, o_ref):
    x = x_ref[...]
    h = jnp.dot(x, w1_ref[...], preferred_element_type=jnp.float32) + b1_ref[...]
    h = jnp.maximum(h, 0.0)
    h = jnp.dot(h, w2_ref[...], preferred_element_type=jnp.float32) + b2_ref[...]
    h = jnp.maximum(h, 0.0)
    o_ref[...] = (
        jnp.dot(h, w3_ref[...], preferred_element_type=jnp.float32) + b3_ref[...]
    )


def _mlp(pooled, W1s, b1, W2, b2, W3, b3):
    return pl.pallas_call(
        _mlp_body,
        grid=(B // BM,),
        in_specs=[
            pl.BlockSpec((BM, EMB), lambda i: (i, 0)),
            pl.BlockSpec((EMB, HID), lambda i: (0, 0)),
            pl.BlockSpec((1, HID), lambda i: (0, 0)),
            pl.BlockSpec((HID, HID), lambda i: (0, 0)),
            pl.BlockSpec((1, HID), lambda i: (0, 0)),
            pl.BlockSpec((HID, OUT), lambda i: (0, 0)),
            pl.BlockSpec((1, OUT), lambda i: (0, 0)),
        ],
        out_specs=pl.BlockSpec((BM, OUT), lambda i: (i, 0)),
        out_shape=jax.ShapeDtypeStruct((B, OUT), jnp.float32),
    )(pooled, W1s, b1, W2, b2, W3, b3)


def kernel(text, table, W1, b1, W2, b2, W3, b3):
    pooled_sum = _pool(text, table)
    W1s = W1 * jnp.float32(1.0 / L)
    return _mlp(
        pooled_sum,
        W1s,
        b1.reshape(1, HID),
        W2,
        b2.reshape(1, HID),
        W3,
        b3.reshape(1, OUT),
    )
